# baseline (speedup 1.0000x reference)
import functools

import jax
import jax.numpy as jnp
from jax import lax
from jax.experimental import pallas as pl
from jax.experimental.pallas import tpu as pltpu
from jax.experimental.pallas import tpu_sc as plsc

_NW = 16
_CH = 16384
_L = 16
_U = 16


def _sc_body(sc_base, per_w, yp_hbm, s_hbm, g_hbm, out_hbm,
             ypA, sA, gA, ypB, sB, gB, outb, semA, semB):
    wid = lax.axis_index("s")
    base = sc_base + wid * per_w
    n_chunks = per_w // _CH
    n_pairs = n_chunks // 2

    def start(bufs, sem, off):
        pltpu.async_copy(yp_hbm.at[pl.ds(off, _CH)], bufs[0], sem)
        pltpu.async_copy(s_hbm.at[pl.ds(off, _CH)], bufs[1], sem)
        pltpu.async_copy(g_hbm.at[pl.ds(off, _CH)], bufs[2], sem)

    def drain(bufs, sem):
        pltpu.make_async_copy(yp_hbm.at[pl.ds(0, _CH)], bufs[0], sem).wait()
        pltpu.make_async_copy(s_hbm.at[pl.ds(0, _CH)], bufs[1], sem).wait()
        pltpu.make_async_copy(g_hbm.at[pl.ds(0, _CH)], bufs[2], sem).wait()

    bufA = (ypA, sA, gA)
    bufB = (ypB, sB, gB)

    def process(bufs, accs):
        ypb, sb, gb = bufs

        def inner(j, acc):
            acc = list(acc)
            jb = j * (_U * _L)
            for k in range(_U):
                o = jb + k * _L
                yp = ypb[pl.ds(o, _L)]
                sv = sb[pl.ds(o, _L)]
                gv = gb[pl.ds(o, _L)]
                gs = gv & sv
                gf = gv.astype(jnp.float32)
                gsf = gs.astype(jnp.float32)
                t = 4 * (k % 2)
                acc[t + 0] = acc[t + 0] + gf * yp
                acc[t + 1] = acc[t + 1] + gsf * yp
                acc[t + 2] = acc[t + 2] + gv
                acc[t + 3] = acc[t + 3] + gs
            return tuple(acc)

        return lax.fori_loop(0, _CH // (_U * _L), inner, accs)

    zf = jnp.zeros((_L,), jnp.float32)
    zi = jnp.zeros((_L,), jnp.int32)
    accs = (zf, zf, zi, zi, zf, zf, zi, zi)

    start(bufA, semA, base)

    def pair_body(cp, accs):
        c0 = 2 * cp
        start(bufB, semB, base + (c0 + 1) * _CH)
        drain(bufA, semA)
        accs = process(bufA, accs)

        @pl.when(c0 + 2 < n_chunks)
        def _():
            start(bufA, semA, base + (c0 + 2) * _CH)

        drain(bufB, semB)
        return process(bufB, accs)

    accs = lax.fori_loop(0, n_pairs, pair_body, accs)

    sp = accs[0] + accs[4]
    s1 = accs[1] + accs[5]
    npos = (accs[2] + accs[6]).astype(jnp.float32)
    n1 = (accs[3] + accs[7]).astype(jnp.float32)
    outb[pl.ds(0, _L)] = sp
    outb[pl.ds(_L, _L)] = s1
    outb[pl.ds(2 * _L, _L)] = npos
    outb[pl.ds(3 * _L, _L)] = n1
    pltpu.sync_copy(outb, out_hbm.at[wid])


def _sc_partials_1core(yp, sv, gv, sc_base, sc_elems):
    per_w = sc_elems // _NW
    mesh = plsc.VectorSubcoreMesh(
        core_axis_name="c", subcore_axis_name="s", num_cores=1)
    k = functools.partial(
        pl.kernel,
        mesh=mesh,
        out_type=jax.ShapeDtypeStruct((_NW, 4 * _L), jnp.float32),
        scratch_types=[
            pltpu.VMEM((_CH,), jnp.float32),
            pltpu.VMEM((_CH,), jnp.int32),
            pltpu.VMEM((_CH,), jnp.int32),
            pltpu.VMEM((_CH,), jnp.float32),
            pltpu.VMEM((_CH,), jnp.int32),
            pltpu.VMEM((_CH,), jnp.int32),
            pltpu.VMEM((4 * _L,), jnp.float32),
            pltpu.SemaphoreType.DMA,
            pltpu.SemaphoreType.DMA,
        ],
    )(functools.partial(_sc_body, sc_base, per_w))
    return k(yp, sv, gv)


def kernel(y_pred, s, y_gt):
    yp = y_pred.reshape(-1)
    sv = s.astype(jnp.int32).reshape(-1)
    gv = y_gt.astype(jnp.int32).reshape(-1)
    n = yp.size
    half = n // 2

    pa = _sc_partials_1core(yp, sv, gv, 0, half)
    pb = _sc_partials_1core(yp, sv, gv, half, half)

    p = (jnp.sum(pa.reshape(_NW, 4, _L), axis=(0, 2))
         + jnp.sum(pb.reshape(_NW, 4, _L), axis=(0, 2)))
    sumpos, sum1, npos, n1 = p[0], p[1], p[2], p[3]
    sum0 = sumpos - sum1
    n0 = npos - n1
    mean0 = sum0 / jnp.maximum(n0, jnp.float32(1.0))
    mean1 = sum1 / jnp.maximum(n1, jnp.float32(1.0))
    loss = jnp.abs(mean0 - mean1)
    return jnp.where((n0 == 0.0) | (n1 == 0.0), jnp.float32(0.0), loss)


# speedup vs baseline: 3.8320x; 3.8320x over previous
import jax
import jax.numpy as jnp
from jax.experimental import pallas as pl
from jax.experimental.pallas import tpu as pltpu

_COLS = 128
_ROWS_PER_BLOCK = 8192


def _body(yp_ref, s_ref, g_ref, out_ref, s1_ref, sp_ref, n1_ref, np_ref):
    i = pl.program_id(0)
    k = pl.num_programs(0)

    yp = yp_ref[...]
    gv = g_ref[...]
    gs = gv & s_ref[...]
    gf = gv.astype(jnp.float32)
    gsf = gs.astype(jnp.float32)

    def chunk_sum(x):
        return jnp.sum(x.reshape(-1, 8, 128), axis=0)

    p_sp = chunk_sum(gf * yp)
    p_s1 = chunk_sum(gsf * yp)
    p_np = chunk_sum(gf)
    p_n1 = chunk_sum(gsf)

    @pl.when(i == 0)
    def _init():
        sp_ref[...] = p_sp
        s1_ref[...] = p_s1
        np_ref[...] = p_np
        n1_ref[...] = p_n1

    @pl.when(i > 0)
    def _acc():
        sp_ref[...] += p_sp
        s1_ref[...] += p_s1
        np_ref[...] += p_np
        n1_ref[...] += p_n1

    @pl.when(i == k - 1)
    def _fini():
        sum1 = jnp.sum(s1_ref[...])
        sumpos = jnp.sum(sp_ref[...])
        n1 = jnp.sum(n1_ref[...])
        npos = jnp.sum(np_ref[...])
        sum0 = sumpos - sum1
        n0 = npos - n1
        mean0 = sum0 / jnp.maximum(n0, jnp.float32(1.0))
        mean1 = sum1 / jnp.maximum(n1, jnp.float32(1.0))
        loss = jnp.abs(mean0 - mean1)
        out_ref[0] = jnp.where((n0 == 0.0) | (n1 == 0.0), jnp.float32(0.0), loss)


def kernel(y_pred, s, y_gt):
    n = y_pred.size
    rows = n // _COLS
    grid = rows // _ROWS_PER_BLOCK
    yp = y_pred.reshape(rows, _COLS)
    sv = s.astype(jnp.int32).reshape(rows, _COLS)
    gv = y_gt.astype(jnp.int32).reshape(rows, _COLS)

    blk = (_ROWS_PER_BLOCK, _COLS)
    in_spec = pl.BlockSpec(blk, lambda i: (i, 0))
    out = pl.pallas_call(
        _body,
        grid=(grid,),
        in_specs=[in_spec, in_spec, in_spec],
        out_specs=pl.BlockSpec(memory_space=pltpu.SMEM),
        out_shape=jax.ShapeDtypeStruct((1,), jnp.float32),
        scratch_shapes=[
            pltpu.VMEM((8, 128), jnp.float32),
            pltpu.VMEM((8, 128), jnp.float32),
            pltpu.VMEM((8, 128), jnp.float32),
            pltpu.VMEM((8, 128), jnp.float32),
        ],
    )(yp, sv, gv)
    return out[0]


# speedup vs baseline: 3.9323x; 1.0262x over previous
import jax
import jax.numpy as jnp
from jax.experimental import pallas as pl
from jax.experimental.pallas import tpu as pltpu

_COLS = 128
_ROWS_PER_BLOCK = 4096


def _body(yp_ref, s_ref, g_ref, out_ref, s1_ref, sp_ref, n1_ref, np_ref):
    i = pl.program_id(0)
    k = pl.num_programs(0)

    yp = yp_ref[...]
    gv = g_ref[...]
    gs = gv & s_ref[...]
    gf = gv.astype(jnp.float32)
    gsf = gs.astype(jnp.float32)

    def chunk_sum(x):
        return jnp.sum(x.reshape(-1, 8, 128), axis=0)

    p_sp = chunk_sum(gf * yp)
    p_s1 = chunk_sum(gsf * yp)
    p_np = chunk_sum(gf)
    p_n1 = chunk_sum(gsf)

    @pl.when(i == 0)
    def _init():
        sp_ref[...] = p_sp
        s1_ref[...] = p_s1
        np_ref[...] = p_np
        n1_ref[...] = p_n1

    @pl.when(i > 0)
    def _acc():
        sp_ref[...] += p_sp
        s1_ref[...] += p_s1
        np_ref[...] += p_np
        n1_ref[...] += p_n1

    @pl.when(i == k - 1)
    def _fini():
        sum1 = jnp.sum(s1_ref[...])
        sumpos = jnp.sum(sp_ref[...])
        n1 = jnp.sum(n1_ref[...])
        npos = jnp.sum(np_ref[...])
        sum0 = sumpos - sum1
        n0 = npos - n1
        mean0 = sum0 / jnp.maximum(n0, jnp.float32(1.0))
        mean1 = sum1 / jnp.maximum(n1, jnp.float32(1.0))
        loss = jnp.abs(mean0 - mean1)
        out_ref[0] = jnp.where((n0 == 0.0) | (n1 == 0.0), jnp.float32(0.0), loss)


def kernel(y_pred, s, y_gt):
    n = y_pred.size
    rows = n // _COLS
    grid = rows // _ROWS_PER_BLOCK
    yp = y_pred.reshape(rows, _COLS)
    sv = s.astype(jnp.int32).reshape(rows, _COLS)
    gv = y_gt.astype(jnp.int32).reshape(rows, _COLS)

    blk = (_ROWS_PER_BLOCK, _COLS)
    in_spec = pl.BlockSpec(blk, lambda i: (i, 0))
    out = pl.pallas_call(
        _body,
        grid=(grid,),
        in_specs=[in_spec, in_spec, in_spec],
        out_specs=pl.BlockSpec(memory_space=pltpu.SMEM),
        out_shape=jax.ShapeDtypeStruct((1,), jnp.float32),
        scratch_shapes=[
            pltpu.VMEM((8, 128), jnp.float32),
            pltpu.VMEM((8, 128), jnp.float32),
            pltpu.VMEM((8, 128), jnp.float32),
            pltpu.VMEM((8, 128), jnp.float32),
        ],
    )(yp, sv, gv)
    return out[0]
